# Initial kernel scaffold; baseline (speedup 1.0000x reference)
import jax
import jax.numpy as jnp
from jax.experimental import pallas as pl


def kernel(x, edge_index, W1_0, W1_1, b1, ln_g, ln_b, W2_0, W2_1, b2, dec_W, dec_b, pr_W1, pr_b1, pr_W2, pr_b2):
    raise NotImplementedError("write your pallas kernel here")


# speedup vs baseline: 10.1201x; 10.1201x over previous
import functools

import jax
import jax.numpy as jnp
from jax import lax
from jax.experimental import pallas as pl
from jax.experimental.pallas import tpu as pltpu
from jax.experimental.pallas import tpu_sc as plsc

N = 10000
E = 320000
D = 128

NC = 2
NS = 16
EPC = E // NC
EPT = EPC // NS
CH = 80
NCH = EPT // CH
DEGW = 16
ZR = 1000

_mesh = plsc.VectorSubcoreMesh(core_axis_name="c", subcore_axis_name="s")


def _deg_body(src_hbm, ones_hbm, zeros_hbm, out_hbm, acc, ones_v, idx_v):
  c = lax.axis_index("c")
  s = lax.axis_index("s")

  @pl.when(s < N // ZR)
  def _zero():
    pltpu.sync_copy(zeros_hbm, acc.at[pl.ds(s * ZR, ZR)])

  pltpu.sync_copy(ones_hbm, ones_v)
  plsc.subcore_barrier()

  base = (c * NS + s) * EPT

  def step(i, carry):
    pltpu.sync_copy(src_hbm.at[pl.ds(base + i * CH, CH)], idx_v)
    pltpu.sync_copy(ones_v, acc.at[idx_v], add=True)
    return carry

  lax.fori_loop(0, NCH, step, 0)
  plsc.subcore_barrier()

  @pl.when(s < N // ZR)
  def _out():
    pltpu.sync_copy(acc.at[pl.ds(s * ZR, ZR)], out_hbm.at[c, pl.ds(s * ZR, ZR)])


def _deg_call(src, ones128, zeros128):
  return pl.kernel(
      _deg_body,
      out_type=jax.ShapeDtypeStruct((NC, N, D), jnp.float32),
      mesh=_mesh,
      scratch_types=[
          pltpu.VMEM_SHARED((N, D), jnp.float32),
          pltpu.VMEM((CH, D), jnp.float32),
          pltpu.VMEM((CH,), jnp.int32),
      ],
  )(src, ones128, zeros128)


def _segsum_body(src_hbm, dst_hbm, table_hbm, zeros_hbm, out_hbm,
                 acc, sidx, didx, rows, sem):
  c = lax.axis_index("c")
  s = lax.axis_index("s")

  @pl.when(s < N // ZR)
  def _zero():
    pltpu.sync_copy(zeros_hbm, acc.at[pl.ds(s * ZR, ZR)])

  plsc.subcore_barrier()

  base = (c * NS + s) * EPT

  def step(i, carry):
    off = base + i * CH
    pltpu.sync_copy(src_hbm.at[pl.ds(off, CH)], sidx)
    pltpu.sync_copy(dst_hbm.at[pl.ds(off, CH)], didx)
    pltpu.async_copy(table_hbm.at[sidx], rows, sem).wait()
    pltpu.sync_copy(rows, acc.at[didx], add=True)
    return carry

  lax.fori_loop(0, NCH, step, 0)
  plsc.subcore_barrier()

  @pl.when(s < N // ZR)
  def _out():
    pltpu.sync_copy(acc.at[pl.ds(s * ZR, ZR)], out_hbm.at[c, pl.ds(s * ZR, ZR)])


def _segsum_call(src, dst, table, zeros128):
  return pl.kernel(
      _segsum_body,
      out_type=jax.ShapeDtypeStruct((NC, N, D), jnp.float32),
      mesh=_mesh,
      scratch_types=[
          pltpu.VMEM_SHARED((N, D), jnp.float32),
          pltpu.VMEM((CH,), jnp.int32),
          pltpu.VMEM((CH,), jnp.int32),
          pltpu.VMEM((CH, D), jnp.float32),
          pltpu.SemaphoreType.DMA,
      ],
  )(src, dst, table, zeros128)


R = 1000
_GRID = N // R


def _dis_from(degp_ref):
  deg = degp_ref[0, :, 0] + degp_ref[1, :, 0]
  return jnp.where(deg > 0, lax.rsqrt(deg), 0.0)


def _tc_a_body(degp_ref, x_ref, w_ref, y1_ref, xw_ref):
  dis = _dis_from(degp_ref)
  xb = x_ref[...]
  y1_ref[...] = xb * dis[:, None]
  xw_ref[...] = jnp.dot(xb, w_ref[...], preferred_element_type=jnp.float32)


def _tc_a(degp, x, w10):
  return pl.pallas_call(
      _tc_a_body,
      grid=(_GRID,),
      in_specs=[
          pl.BlockSpec((NC, R, D), lambda i: (0, i, 0)),
          pl.BlockSpec((R, D), lambda i: (i, 0)),
          pl.BlockSpec((D, D), lambda i: (0, 0)),
      ],
      out_specs=[
          pl.BlockSpec((R, D), lambda i: (i, 0)),
          pl.BlockSpec((R, D), lambda i: (i, 0)),
      ],
      out_shape=[
          jax.ShapeDtypeStruct((N, D), jnp.float32),
          jax.ShapeDtypeStruct((N, D), jnp.float32),
      ],
  )(degp, x, w10)


def _tc_b_body(degp_ref, xw_ref, p_ref, w11_ref, b1_ref, g_ref, bb_ref,
               w20_ref, y2_ref, hw_ref):
  dis = _dis_from(degp_ref)
  tx = (p_ref[0] + p_ref[1]) * (-dis[:, None])
  h = (xw_ref[...] + b1_ref[...]
       + jnp.dot(tx, w11_ref[...], preferred_element_type=jnp.float32))
  mu = jnp.mean(h, axis=1, keepdims=True)
  hc = h - mu
  var = jnp.mean(hc * hc, axis=1, keepdims=True)
  hn = hc * lax.rsqrt(var + 1e-5) * g_ref[...] + bb_ref[...]
  h = jnp.where(hn >= 0, hn, 0.01 * hn)
  y2_ref[...] = h * dis[:, None]
  hw_ref[...] = jnp.dot(h, w20_ref[...], preferred_element_type=jnp.float32)


def _tc_b(degp, xw, p, w11, b1, ln_g, ln_b, w20):
  wspec = pl.BlockSpec((D, D), lambda i: (0, 0))
  vspec = pl.BlockSpec((1, D), lambda i: (0, 0))
  rspec = pl.BlockSpec((R, D), lambda i: (i, 0))
  return pl.pallas_call(
      _tc_b_body,
      grid=(_GRID,),
      in_specs=[
          pl.BlockSpec((NC, R, D), lambda i: (0, i, 0)),
          rspec,
          pl.BlockSpec((NC, R, D), lambda i: (0, i, 0)),
          wspec, vspec, vspec, vspec, wspec,
      ],
      out_specs=[rspec, rspec],
      out_shape=[
          jax.ShapeDtypeStruct((N, D), jnp.float32),
          jax.ShapeDtypeStruct((N, D), jnp.float32),
      ],
  )(degp, xw, p, w11, b1.reshape(1, D), ln_g.reshape(1, D),
    ln_b.reshape(1, D), w20)


def _tc_c_body(degp_ref, hw_ref, q_ref, w21_ref, b2_ref, decw_ref, decb_ref,
               pw1_ref, pb1_ref, pw2_ref, pb2_ref, recon_ref, z_ref, proj_ref):
  dis = _dis_from(degp_ref)
  tx = (q_ref[0] + q_ref[1]) * (-dis[:, None])
  z = (hw_ref[...] + b2_ref[...]
       + jnp.dot(tx, w21_ref[...], preferred_element_type=jnp.float32))
  z_ref[...] = z
  recon_ref[...] = jnp.tanh(
      jnp.dot(z, decw_ref[...], preferred_element_type=jnp.float32)
      + decb_ref[...])
  t = jnp.maximum(
      jnp.dot(z, pw1_ref[...], preferred_element_type=jnp.float32)
      + pb1_ref[...], 0.0)
  proj_ref[...] = (
      jnp.dot(t, pw2_ref[...], preferred_element_type=jnp.float32)
      + pb2_ref[...])


def _tc_c(degp, hw, q, w21, b2, dec_w, dec_b, pw1, pb1, pw2, pb2):
  wspec = pl.BlockSpec((D, D), lambda i: (0, 0))
  vspec = pl.BlockSpec((1, D), lambda i: (0, 0))
  rspec = pl.BlockSpec((R, D), lambda i: (i, 0))
  return pl.pallas_call(
      _tc_c_body,
      grid=(_GRID,),
      in_specs=[
          pl.BlockSpec((NC, R, D), lambda i: (0, i, 0)),
          rspec,
          pl.BlockSpec((NC, R, D), lambda i: (0, i, 0)),
          wspec, vspec, wspec, vspec, wspec, vspec, wspec, vspec,
      ],
      out_specs=[rspec, rspec, rspec],
      out_shape=[
          jax.ShapeDtypeStruct((N, D), jnp.float32),
          jax.ShapeDtypeStruct((N, D), jnp.float32),
          jax.ShapeDtypeStruct((N, D), jnp.float32),
      ],
  )(degp, hw, q, w21, b2.reshape(1, D), dec_w, dec_b.reshape(1, D),
    pw1, pb1.reshape(1, D), pw2, pb2.reshape(1, D))


def kernel(x, edge_index, W1_0, W1_1, b1, ln_g, ln_b, W2_0, W2_1, b2,
           dec_W, dec_b, pr_W1, pr_b1, pr_W2, pr_b2):
  src = edge_index[0]
  dst = edge_index[1]
  zeros128 = jnp.zeros((ZR, D), jnp.float32)
  ones128 = jnp.ones((CH, D), jnp.float32)

  degp = _deg_call(src, ones128, zeros128)
  y1, xw0 = _tc_a(degp, x, W1_0)
  p = _segsum_call(src, dst, y1, zeros128)
  y2, hw0 = _tc_b(degp, xw0, p, W1_1, b1, ln_g, ln_b, W2_0)
  q = _segsum_call(src, dst, y2, zeros128)
  recon, z, proj = _tc_c(degp, hw0, q, W2_1, b2, dec_W, dec_b,
                         pr_W1, pr_b1, pr_W2, pr_b2)
  return (recon, z, proj)


# speedup vs baseline: 18.0903x; 1.7876x over previous
import functools

import jax
import jax.numpy as jnp
from jax import lax
from jax.experimental import pallas as pl
from jax.experimental.pallas import tpu as pltpu
from jax.experimental.pallas import tpu_sc as plsc

N = 10000
E = 320000
D = 128

NC = 2
NS = 16
NW = NC * NS
CH = 128
NCHT = E // CH // NW
TAIL = E // CH - NCHT * NW
WSTRIDE = 80
IDXR = WSTRIDE
HIDX = 40
NBUF = 2
ZR = 1000

_mesh = plsc.VectorSubcoreMesh(core_axis_name="c", subcore_axis_name="s")


def _wid_start(c, s):
  wid = c * NS + s
  return wid, WSTRIDE * wid


def _chunk_layout(v2d):
  main = v2d[:NCHT * NW].reshape(NW, NCHT, CH)
  main = jnp.pad(main, ((0, 0), (0, WSTRIDE - NCHT), (0, 0)))
  out = main.reshape(NW * WSTRIDE, CH)
  tail = v2d[NCHT * NW:]
  rows = NCHT + WSTRIDE * jnp.arange(TAIL)
  return out.at[rows].set(tail)


def _deg_body(src2_hbm, ones_hbm, zeros_hbm, out_hbm, acc, ones_v, sidx, ssem):
  c = lax.axis_index("c")
  s = lax.axis_index("s")
  wid, start = _wid_start(c, s)

  @pl.when(s < N // ZR)
  def _zero():
    pltpu.sync_copy(zeros_hbm, acc.at[pl.ds(s * ZR, ZR)])

  pltpu.sync_copy(ones_hbm, ones_v)
  pltpu.sync_copy(src2_hbm.at[pl.ds(start, IDXR)], sidx)
  plsc.subcore_barrier()

  def blk(i, carry):
    for b in range(NBUF):
      pltpu.async_copy(ones_v, acc.at[sidx.at[i * NBUF + b]], ssem.at[b],
                       add=True)
    for b in range(NBUF):
      pltpu.make_async_copy(ones_v, acc.at[sidx.at[i * NBUF + b]],
                            ssem.at[b]).wait()
    return carry

  lax.fori_loop(0, NCHT // NBUF, blk, 0)

  @pl.when(wid < TAIL)
  def _tail():
    pltpu.sync_copy(ones_v, acc.at[sidx.at[NCHT]], add=True)

  plsc.subcore_barrier()

  @pl.when(s < N // ZR)
  def _out():
    pltpu.sync_copy(acc.at[pl.ds(s * ZR, ZR)], out_hbm.at[c, pl.ds(s * ZR, ZR)])


def _deg_call(src2, ones128, zeros128):
  return pl.kernel(
      _deg_body,
      out_type=jax.ShapeDtypeStruct((NC, N, D), jnp.float32),
      mesh=_mesh,
      scratch_types=[
          pltpu.VMEM_SHARED((N, D), jnp.float32),
          pltpu.VMEM((CH, D), jnp.float32),
          pltpu.VMEM((IDXR, CH), jnp.int32),
          pltpu.SemaphoreType.DMA((NBUF,)),
      ],
  )(src2, ones128, zeros128)


def _segsum_body(src2_hbm, dst2_hbm, table_hbm, zeros_hbm, out_hbm,
                 acc, sidx, didx, rows, gsem, ssem):
  c = lax.axis_index("c")
  s = lax.axis_index("s")
  wid, start = _wid_start(c, s)

  @pl.when(s < N // ZR)
  def _zero():
    pltpu.sync_copy(zeros_hbm, acc.at[pl.ds(s * ZR, ZR)])

  plsc.subcore_barrier()

  def phase(p0, cnt):
    pltpu.sync_copy(src2_hbm.at[pl.ds(start + p0, HIDX)], sidx)
    pltpu.sync_copy(dst2_hbm.at[pl.ds(start + p0, HIDX)], didx)
    for b in range(NBUF):
      pltpu.async_copy(table_hbm.at[sidx.at[b]], rows.at[b], gsem.at[b])

    nblk = cnt // NBUF

    def blk(i, carry):
      for b in range(NBUF):
        j = i * NBUF + b
        pltpu.make_async_copy(table_hbm.at[sidx.at[j]], rows.at[b],
                              gsem.at[b]).wait()
        pltpu.async_copy(rows.at[b], acc.at[didx.at[j]], ssem.at[b], add=True)
      for b in range(NBUF):
        j = i * NBUF + b
        pltpu.make_async_copy(rows.at[b], acc.at[didx.at[j]],
                              ssem.at[b]).wait()

        @pl.when(i < nblk - 1)
        def _pref():
          pltpu.async_copy(table_hbm.at[sidx.at[j + NBUF]], rows.at[b],
                           gsem.at[b])

      return carry

    lax.fori_loop(0, nblk, blk, 0)

  phase(0, HIDX)
  phase(HIDX, NCHT - HIDX)

  @pl.when(wid < TAIL)
  def _tail():
    pltpu.async_copy(table_hbm.at[sidx.at[NCHT - HIDX]], rows.at[0],
                     gsem.at[0]).wait()
    pltpu.sync_copy(rows.at[0], acc.at[didx.at[NCHT - HIDX]], add=True)

  plsc.subcore_barrier()

  @pl.when(s < N // ZR)
  def _out():
    pltpu.sync_copy(acc.at[pl.ds(s * ZR, ZR)], out_hbm.at[c, pl.ds(s * ZR, ZR)])


def _segsum_call(src2, dst2, table, zeros128):
  return pl.kernel(
      _segsum_body,
      out_type=jax.ShapeDtypeStruct((NC, N, D), jnp.float32),
      mesh=_mesh,
      scratch_types=[
          pltpu.VMEM_SHARED((N, D), jnp.float32),
          pltpu.VMEM((HIDX, CH), jnp.int32),
          pltpu.VMEM((HIDX, CH), jnp.int32),
          pltpu.VMEM((NBUF, CH, D), jnp.float32),
          pltpu.SemaphoreType.DMA((NBUF,)),
          pltpu.SemaphoreType.DMA((NBUF,)),
      ],
  )(src2, dst2, table, zeros128)


R = 1000
_GRID = N // R


def _dis_from(degp_ref):
  deg = degp_ref[0, :, 0] + degp_ref[1, :, 0]
  return jnp.where(deg > 0, lax.rsqrt(deg), 0.0)


def _tc_a_body(degp_ref, x_ref, w_ref, y1_ref, xw_ref):
  dis = _dis_from(degp_ref)
  xb = x_ref[...]
  y1_ref[...] = xb * dis[:, None]
  xw_ref[...] = jnp.dot(xb, w_ref[...], preferred_element_type=jnp.float32)


def _tc_a(degp, x, w10):
  return pl.pallas_call(
      _tc_a_body,
      grid=(_GRID,),
      in_specs=[
          pl.BlockSpec((NC, R, D), lambda i: (0, i, 0)),
          pl.BlockSpec((R, D), lambda i: (i, 0)),
          pl.BlockSpec((D, D), lambda i: (0, 0)),
      ],
      out_specs=[
          pl.BlockSpec((R, D), lambda i: (i, 0)),
          pl.BlockSpec((R, D), lambda i: (i, 0)),
      ],
      out_shape=[
          jax.ShapeDtypeStruct((N, D), jnp.float32),
          jax.ShapeDtypeStruct((N, D), jnp.float32),
      ],
  )(degp, x, w10)


def _tc_b_body(degp_ref, xw_ref, p_ref, w11_ref, b1_ref, g_ref, bb_ref,
               w20_ref, y2_ref, hw_ref):
  dis = _dis_from(degp_ref)
  tx = (p_ref[0] + p_ref[1]) * (-dis[:, None])
  h = (xw_ref[...] + b1_ref[...]
       + jnp.dot(tx, w11_ref[...], preferred_element_type=jnp.float32))
  mu = jnp.mean(h, axis=1, keepdims=True)
  hc = h - mu
  var = jnp.mean(hc * hc, axis=1, keepdims=True)
  hn = hc * lax.rsqrt(var + 1e-5) * g_ref[...] + bb_ref[...]
  h = jnp.where(hn >= 0, hn, 0.01 * hn)
  y2_ref[...] = h * dis[:, None]
  hw_ref[...] = jnp.dot(h, w20_ref[...], preferred_element_type=jnp.float32)


def _tc_b(degp, xw, p, w11, b1, ln_g, ln_b, w20):
  wspec = pl.BlockSpec((D, D), lambda i: (0, 0))
  vspec = pl.BlockSpec((1, D), lambda i: (0, 0))
  rspec = pl.BlockSpec((R, D), lambda i: (i, 0))
  return pl.pallas_call(
      _tc_b_body,
      grid=(_GRID,),
      in_specs=[
          pl.BlockSpec((NC, R, D), lambda i: (0, i, 0)),
          rspec,
          pl.BlockSpec((NC, R, D), lambda i: (0, i, 0)),
          wspec, vspec, vspec, vspec, wspec,
      ],
      out_specs=[rspec, rspec],
      out_shape=[
          jax.ShapeDtypeStruct((N, D), jnp.float32),
          jax.ShapeDtypeStruct((N, D), jnp.float32),
      ],
  )(degp, xw, p, w11, b1.reshape(1, D), ln_g.reshape(1, D),
    ln_b.reshape(1, D), w20)


def _tc_c_body(degp_ref, hw_ref, q_ref, w21_ref, b2_ref, decw_ref, decb_ref,
               pw1_ref, pb1_ref, pw2_ref, pb2_ref, recon_ref, z_ref, proj_ref):
  dis = _dis_from(degp_ref)
  tx = (q_ref[0] + q_ref[1]) * (-dis[:, None])
  z = (hw_ref[...] + b2_ref[...]
       + jnp.dot(tx, w21_ref[...], preferred_element_type=jnp.float32))
  z_ref[...] = z
  recon_ref[...] = jnp.tanh(
      jnp.dot(z, decw_ref[...], preferred_element_type=jnp.float32)
      + decb_ref[...])
  t = jnp.maximum(
      jnp.dot(z, pw1_ref[...], preferred_element_type=jnp.float32)
      + pb1_ref[...], 0.0)
  proj_ref[...] = (
      jnp.dot(t, pw2_ref[...], preferred_element_type=jnp.float32)
      + pb2_ref[...])


def _tc_c(degp, hw, q, w21, b2, dec_w, dec_b, pw1, pb1, pw2, pb2):
  wspec = pl.BlockSpec((D, D), lambda i: (0, 0))
  vspec = pl.BlockSpec((1, D), lambda i: (0, 0))
  rspec = pl.BlockSpec((R, D), lambda i: (i, 0))
  return pl.pallas_call(
      _tc_c_body,
      grid=(_GRID,),
      in_specs=[
          pl.BlockSpec((NC, R, D), lambda i: (0, i, 0)),
          rspec,
          pl.BlockSpec((NC, R, D), lambda i: (0, i, 0)),
          wspec, vspec, wspec, vspec, wspec, vspec, wspec, vspec,
      ],
      out_specs=[rspec, rspec, rspec],
      out_shape=[
          jax.ShapeDtypeStruct((N, D), jnp.float32),
          jax.ShapeDtypeStruct((N, D), jnp.float32),
          jax.ShapeDtypeStruct((N, D), jnp.float32),
      ],
  )(degp, hw, q, w21, b2.reshape(1, D), dec_w, dec_b.reshape(1, D),
    pw1, pb1.reshape(1, D), pw2, pb2.reshape(1, D))


def kernel(x, edge_index, W1_0, W1_1, b1, ln_g, ln_b, W2_0, W2_1, b2,
           dec_W, dec_b, pr_W1, pr_b1, pr_W2, pr_b2):
  src2 = _chunk_layout(edge_index[0].reshape(E // CH, CH))
  dst2 = _chunk_layout(edge_index[1].reshape(E // CH, CH))
  zeros128 = jnp.zeros((ZR, D), jnp.float32)
  ones128 = jnp.ones((CH, D), jnp.float32)

  degp = _deg_call(src2, ones128, zeros128)
  y1, xw0 = _tc_a(degp, x, W1_0)
  p = _segsum_call(src2, dst2, y1, zeros128)
  y2, hw0 = _tc_b(degp, xw0, p, W1_1, b1, ln_g, ln_b, W2_0)
  q = _segsum_call(src2, dst2, y2, zeros128)
  recon, z, proj = _tc_c(degp, hw0, q, W2_1, b2, dec_W, dec_b,
                         pr_W1, pr_b1, pr_W2, pr_b2)
  return (recon, z, proj)


# speedup vs baseline: 18.0963x; 1.0003x over previous
import functools

import jax
import jax.numpy as jnp
from jax import lax
from jax.experimental import pallas as pl
from jax.experimental.pallas import tpu as pltpu
from jax.experimental.pallas import tpu_sc as plsc

N = 10000
E = 320000
D = 128

NC = 2
NS = 16
NW = NC * NS
CH = 128
NCHT = E // CH // NW
TAIL = E // CH - NCHT * NW
WSTRIDE = 80
IDXR = WSTRIDE
HIDX = 40
NBUF = 2
ZR = 1000
DEGW = 16
NPAD = 10240

_mesh = plsc.VectorSubcoreMesh(core_axis_name="c", subcore_axis_name="s")


def _wid_start(c, s):
  wid = c * NS + s
  return wid, WSTRIDE * wid


def _chunk_layout(v2d):
  main = v2d[:NCHT * NW].reshape(NW, NCHT, CH)
  main = jnp.pad(main, ((0, 0), (0, WSTRIDE - NCHT), (0, 0)))
  out = main.reshape(NW * WSTRIDE, CH)
  tail = v2d[NCHT * NW:]
  rows = NCHT + WSTRIDE * jnp.arange(TAIL)
  return out.at[rows].set(tail)


def _deg_body(src2_hbm, ones_hbm, zeros_hbm, out_hbm, acc, ones_v, sidx, ssem):
  c = lax.axis_index("c")
  s = lax.axis_index("s")
  wid, start = _wid_start(c, s)

  @pl.when(s < N // ZR)
  def _zero():
    pltpu.sync_copy(zeros_hbm, acc.at[pl.ds(s * ZR, ZR)])

  pltpu.sync_copy(ones_hbm, ones_v)
  pltpu.sync_copy(src2_hbm.at[pl.ds(start, IDXR)], sidx)
  plsc.subcore_barrier()

  def blk(i, carry):
    for b in range(NBUF):
      pltpu.async_copy(ones_v, acc.at[sidx.at[i * NBUF + b]], ssem.at[b],
                       add=True)
    for b in range(NBUF):
      pltpu.make_async_copy(ones_v, acc.at[sidx.at[i * NBUF + b]],
                            ssem.at[b]).wait()
    return carry

  lax.fori_loop(0, NCHT // NBUF, blk, 0)

  @pl.when(wid < TAIL)
  def _tail():
    pltpu.sync_copy(ones_v, acc.at[sidx.at[NCHT]], add=True)

  plsc.subcore_barrier()

  @pl.when(s < N // ZR)
  def _out():
    pltpu.sync_copy(acc.at[pl.ds(s * ZR, ZR)], out_hbm.at[c, pl.ds(s * ZR, ZR)])


def _deg_call(src2, ones128, zeros128):
  return pl.kernel(
      _deg_body,
      out_type=jax.ShapeDtypeStruct((NC, N, D), jnp.float32),
      mesh=_mesh,
      scratch_types=[
          pltpu.VMEM_SHARED((N, D), jnp.float32),
          pltpu.VMEM((CH, D), jnp.float32),
          pltpu.VMEM((IDXR, CH), jnp.int32),
          pltpu.SemaphoreType.DMA((NBUF,)),
      ],
  )(src2, ones128, zeros128)


def _segsum_body(src2_hbm, dst2_hbm, table_hbm, zeros_hbm, out_hbm,
                 acc, sidx, didx, rows, gsem, ssem):
  c = lax.axis_index("c")
  s = lax.axis_index("s")
  wid, start = _wid_start(c, s)

  @pl.when(s < N // ZR)
  def _zero():
    pltpu.sync_copy(zeros_hbm, acc.at[pl.ds(s * ZR, ZR)])

  plsc.subcore_barrier()

  def phase(p0, cnt):
    pltpu.sync_copy(src2_hbm.at[pl.ds(start + p0, HIDX)], sidx)
    pltpu.sync_copy(dst2_hbm.at[pl.ds(start + p0, HIDX)], didx)
    for b in range(NBUF):
      pltpu.async_copy(table_hbm.at[sidx.at[b]], rows.at[b], gsem.at[b])

    nblk = cnt // NBUF

    def blk(i, carry):
      for b in range(NBUF):
        j = i * NBUF + b
        pltpu.make_async_copy(table_hbm.at[sidx.at[j]], rows.at[b],
                              gsem.at[b]).wait()
        pltpu.async_copy(rows.at[b], acc.at[didx.at[j]], ssem.at[b], add=True)
      for b in range(NBUF):
        j = i * NBUF + b
        pltpu.make_async_copy(rows.at[b], acc.at[didx.at[j]],
                              ssem.at[b]).wait()

        @pl.when(i < nblk - 1)
        def _pref():
          pltpu.async_copy(table_hbm.at[sidx.at[j + NBUF]], rows.at[b],
                           gsem.at[b])

      return carry

    lax.fori_loop(0, nblk, blk, 0)

  phase(0, HIDX)
  phase(HIDX, NCHT - HIDX)

  @pl.when(wid < TAIL)
  def _tail():
    pltpu.async_copy(table_hbm.at[sidx.at[NCHT - HIDX]], rows.at[0],
                     gsem.at[0]).wait()
    pltpu.sync_copy(rows.at[0], acc.at[didx.at[NCHT - HIDX]], add=True)

  plsc.subcore_barrier()

  @pl.when(s < N // ZR)
  def _out():
    pltpu.sync_copy(acc.at[pl.ds(s * ZR, ZR)], out_hbm.at[c, pl.ds(s * ZR, ZR)])


def _segsum_call(src2, dst2, table, zeros128):
  return pl.kernel(
      _segsum_body,
      out_type=jax.ShapeDtypeStruct((NC, N, D), jnp.float32),
      mesh=_mesh,
      scratch_types=[
          pltpu.VMEM_SHARED((N, D), jnp.float32),
          pltpu.VMEM((HIDX, CH), jnp.int32),
          pltpu.VMEM((HIDX, CH), jnp.int32),
          pltpu.VMEM((NBUF, CH, D), jnp.float32),
          pltpu.SemaphoreType.DMA((NBUF,)),
          pltpu.SemaphoreType.DMA((NBUF,)),
      ],
  )(src2, dst2, table, zeros128)


R = 1000
_GRID = N // R


def _dis_from(degp_ref):
  deg = degp_ref[0, :, 0] + degp_ref[1, :, 0]
  return jnp.where(deg > 0, lax.rsqrt(deg), 0.0)


def _tc_mm_body(x_ref, w_ref, xw_ref):
  xw_ref[...] = jnp.dot(x_ref[...], w_ref[...],
                        preferred_element_type=jnp.float32)


def _tc_mm(x, w):
  return pl.pallas_call(
      _tc_mm_body,
      grid=(_GRID,),
      in_specs=[
          pl.BlockSpec((R, D), lambda i: (i, 0)),
          pl.BlockSpec((D, D), lambda i: (0, 0)),
      ],
      out_specs=pl.BlockSpec((R, D), lambda i: (i, 0)),
      out_shape=jax.ShapeDtypeStruct((N, D), jnp.float32),
  )(x, w)


def _tc_a_body(degp_ref, x_ref, y1_ref):
  dis = _dis_from(degp_ref)
  y1_ref[...] = x_ref[...] * dis[:, None]


def _tc_a(degp, x):
  return pl.pallas_call(
      _tc_a_body,
      grid=(_GRID,),
      in_specs=[
          pl.BlockSpec((NC, R, D), lambda i: (0, i, 0)),
          pl.BlockSpec((R, D), lambda i: (i, 0)),
      ],
      out_specs=pl.BlockSpec((R, D), lambda i: (i, 0)),
      out_shape=jax.ShapeDtypeStruct((N, D), jnp.float32),
  )(degp, x)


def _tc_b_body(degp_ref, xw_ref, p_ref, w11_ref, b1_ref, g_ref, bb_ref,
               w20_ref, y2_ref, hw_ref):
  dis = _dis_from(degp_ref)
  tx = (p_ref[0] + p_ref[1]) * (-dis[:, None])
  h = (xw_ref[...] + b1_ref[...]
       + jnp.dot(tx, w11_ref[...], preferred_element_type=jnp.float32))
  mu = jnp.mean(h, axis=1, keepdims=True)
  hc = h - mu
  var = jnp.mean(hc * hc, axis=1, keepdims=True)
  hn = hc * lax.rsqrt(var + 1e-5) * g_ref[...] + bb_ref[...]
  h = jnp.where(hn >= 0, hn, 0.01 * hn)
  y2_ref[...] = h * dis[:, None]
  hw_ref[...] = jnp.dot(h, w20_ref[...], preferred_element_type=jnp.float32)


def _tc_b(degp, xw, p, w11, b1, ln_g, ln_b, w20):
  wspec = pl.BlockSpec((D, D), lambda i: (0, 0))
  vspec = pl.BlockSpec((1, D), lambda i: (0, 0))
  rspec = pl.BlockSpec((R, D), lambda i: (i, 0))
  return pl.pallas_call(
      _tc_b_body,
      grid=(_GRID,),
      in_specs=[
          pl.BlockSpec((NC, R, D), lambda i: (0, i, 0)),
          rspec,
          pl.BlockSpec((NC, R, D), lambda i: (0, i, 0)),
          wspec, vspec, vspec, vspec, wspec,
      ],
      out_specs=[rspec, rspec],
      out_shape=[
          jax.ShapeDtypeStruct((N, D), jnp.float32),
          jax.ShapeDtypeStruct((N, D), jnp.float32),
      ],
  )(degp, xw, p, w11, b1.reshape(1, D), ln_g.reshape(1, D),
    ln_b.reshape(1, D), w20)


def _tc_c_body(degp_ref, hw_ref, q_ref, w21_ref, b2_ref, decw_ref, decb_ref,
               pw1_ref, pb1_ref, pw2_ref, pb2_ref, recon_ref, z_ref, proj_ref):
  dis = _dis_from(degp_ref)
  tx = (q_ref[0] + q_ref[1]) * (-dis[:, None])
  z = (hw_ref[...] + b2_ref[...]
       + jnp.dot(tx, w21_ref[...], preferred_element_type=jnp.float32))
  z_ref[...] = z
  recon_ref[...] = jnp.tanh(
      jnp.dot(z, decw_ref[...], preferred_element_type=jnp.float32)
      + decb_ref[...])
  t = jnp.maximum(
      jnp.dot(z, pw1_ref[...], preferred_element_type=jnp.float32)
      + pb1_ref[...], 0.0)
  proj_ref[...] = (
      jnp.dot(t, pw2_ref[...], preferred_element_type=jnp.float32)
      + pb2_ref[...])


def _tc_c(degp, hw, q, w21, b2, dec_w, dec_b, pw1, pb1, pw2, pb2):
  wspec = pl.BlockSpec((D, D), lambda i: (0, 0))
  vspec = pl.BlockSpec((1, D), lambda i: (0, 0))
  rspec = pl.BlockSpec((R, D), lambda i: (i, 0))
  return pl.pallas_call(
      _tc_c_body,
      grid=(_GRID,),
      in_specs=[
          pl.BlockSpec((NC, R, D), lambda i: (0, i, 0)),
          rspec,
          pl.BlockSpec((NC, R, D), lambda i: (0, i, 0)),
          wspec, vspec, wspec, vspec, wspec, vspec, wspec, vspec,
      ],
      out_specs=[rspec, rspec, rspec],
      out_shape=[
          jax.ShapeDtypeStruct((N, D), jnp.float32),
          jax.ShapeDtypeStruct((N, D), jnp.float32),
          jax.ShapeDtypeStruct((N, D), jnp.float32),
      ],
  )(degp, hw, q, w21, b2.reshape(1, D), dec_w, dec_b.reshape(1, D),
    pw1, pb1.reshape(1, D), pw2, pb2.reshape(1, D))


def kernel(x, edge_index, W1_0, W1_1, b1, ln_g, ln_b, W2_0, W2_1, b2,
           dec_W, dec_b, pr_W1, pr_b1, pr_W2, pr_b2):
  src2 = _chunk_layout(edge_index[0].reshape(E // CH, CH))
  dst2 = _chunk_layout(edge_index[1].reshape(E // CH, CH))
  zeros128 = jnp.zeros((ZR, D), jnp.float32)
  ones128 = jnp.ones((CH, D), jnp.float32)

  xw0 = _tc_mm(x, W1_0)
  degp = _deg_call(src2, ones128, zeros128)
  y1 = _tc_a(degp, x)
  p = _segsum_call(src2, dst2, y1, zeros128)
  y2, hw0 = _tc_b(degp, xw0, p, W1_1, b1, ln_g, ln_b, W2_0)
  q = _segsum_call(src2, dst2, y2, zeros128)
  recon, z, proj = _tc_c(degp, hw0, q, W2_1, b2, dec_W, dec_b,
                         pr_W1, pr_b1, pr_W2, pr_b2)
  return (recon, z, proj)


# speedup vs baseline: 19.3322x; 1.0683x over previous
import functools

import jax
import jax.numpy as jnp
from jax import lax
from jax.experimental import pallas as pl
from jax.experimental.pallas import tpu as pltpu
from jax.experimental.pallas import tpu_sc as plsc

N = 10000
E = 320000
D = 128

NC = 2
NS = 16
NW = NC * NS
CH = 128
NCHT = E // CH // NW
TAIL = E // CH - NCHT * NW
WSTRIDE = 80
IDXR = WSTRIDE
HIDX = 40
NBUF = 2
ZR = 1000
DEGW = 16
NPAD = 10240

_mesh = plsc.VectorSubcoreMesh(core_axis_name="c", subcore_axis_name="s")


def _wid_start(c, s):
  wid = c * NS + s
  return wid, WSTRIDE * wid


def _chunk_layout(v2d):
  main = v2d[:NCHT * NW].reshape(NW, NCHT, CH)
  main = jnp.pad(main, ((0, 0), (0, WSTRIDE - NCHT), (0, 0)))
  out = main.reshape(NW * WSTRIDE, CH)
  tail = v2d[NCHT * NW:]
  rows = NCHT + WSTRIDE * jnp.arange(TAIL)
  return out.at[rows].set(tail)


def _deg_body(src2_hbm, ones_hbm, zeros_hbm, out_hbm, acc, ones_v, sidx, ssem):
  c = lax.axis_index("c")
  s = lax.axis_index("s")
  wid, start = _wid_start(c, s)

  @pl.when(s < N // ZR)
  def _zero():
    pltpu.sync_copy(zeros_hbm, acc.at[pl.ds(s * ZR, ZR)])

  pltpu.sync_copy(ones_hbm, ones_v)
  pltpu.sync_copy(src2_hbm.at[pl.ds(start, IDXR)], sidx)
  plsc.subcore_barrier()

  def blk(i, carry):
    for b in range(NBUF):
      pltpu.async_copy(ones_v, acc.at[sidx.at[i * NBUF + b]], ssem.at[b],
                       add=True)
    for b in range(NBUF):
      pltpu.make_async_copy(ones_v, acc.at[sidx.at[i * NBUF + b]],
                            ssem.at[b]).wait()
    return carry

  lax.fori_loop(0, NCHT // NBUF, blk, 0)

  @pl.when(wid < TAIL)
  def _tail():
    pltpu.sync_copy(ones_v, acc.at[sidx.at[NCHT]], add=True)

  plsc.subcore_barrier()

  @pl.when(s < N // ZR)
  def _out():
    pltpu.sync_copy(acc.at[pl.ds(s * ZR, ZR)], out_hbm.at[c, pl.ds(s * ZR, ZR)])


def _deg_call(src2, ones128, zeros128):
  return pl.kernel(
      _deg_body,
      out_type=jax.ShapeDtypeStruct((NC, N, D), jnp.float32),
      mesh=_mesh,
      scratch_types=[
          pltpu.VMEM_SHARED((N, D), jnp.float32),
          pltpu.VMEM((CH, D), jnp.float32),
          pltpu.VMEM((IDXR, CH), jnp.int32),
          pltpu.SemaphoreType.DMA((NBUF,)),
      ],
  )(src2, ones128, zeros128)


def _segsum_body(src2_hbm, dst2_hbm, table_hbm, zeros_hbm, out_hbm,
                 acc, sidx, didx, rows, gsem, ssem):
  c = lax.axis_index("c")
  s = lax.axis_index("s")
  wid, start = _wid_start(c, s)

  @pl.when(s < N // ZR)
  def _zero():
    pltpu.sync_copy(zeros_hbm, acc.at[pl.ds(s * ZR, ZR)])

  plsc.subcore_barrier()

  def gwait(j, b):
    pltpu.make_async_copy(table_hbm.at[sidx.at[j]], rows.at[b],
                          gsem.at[b]).wait()

  def swait(j, b):
    pltpu.make_async_copy(rows.at[b], acc.at[didx.at[j]], ssem.at[b]).wait()

  def phase(p0, cnt):
    pltpu.sync_copy(src2_hbm.at[pl.ds(start + p0, HIDX)], sidx)
    pltpu.sync_copy(dst2_hbm.at[pl.ds(start + p0, HIDX)], didx)
    pltpu.async_copy(table_hbm.at[sidx.at[0]], rows.at[0], gsem.at[0])
    gwait(0, 0)
    pltpu.async_copy(rows.at[0], acc.at[didx.at[0]], ssem.at[0], add=True)
    pltpu.async_copy(table_hbm.at[sidx.at[1]], rows.at[1], gsem.at[1])

    def blk(i, carry):
      j = 2 * i + 1
      gwait(j, 1)
      pltpu.async_copy(rows.at[1], acc.at[didx.at[j]], ssem.at[1], add=True)
      swait(j - 1, 0)
      pltpu.async_copy(table_hbm.at[sidx.at[j + 1]], rows.at[0], gsem.at[0])
      gwait(j + 1, 0)
      pltpu.async_copy(rows.at[0], acc.at[didx.at[j + 1]], ssem.at[0],
                       add=True)
      swait(j, 1)
      pltpu.async_copy(table_hbm.at[sidx.at[j + 2]], rows.at[1], gsem.at[1])
      return carry

    lax.fori_loop(0, (cnt - 2) // 2, blk, 0)

    j = cnt - 1
    gwait(j, 1)
    pltpu.async_copy(rows.at[1], acc.at[didx.at[j]], ssem.at[1], add=True)
    swait(j - 1, 0)
    swait(j, 1)

  phase(0, HIDX)
  phase(HIDX, NCHT - HIDX)

  @pl.when(wid < TAIL)
  def _tail():
    pltpu.async_copy(table_hbm.at[sidx.at[NCHT - HIDX]], rows.at[0],
                     gsem.at[0]).wait()
    pltpu.sync_copy(rows.at[0], acc.at[didx.at[NCHT - HIDX]], add=True)

  plsc.subcore_barrier()

  @pl.when(s < N // ZR)
  def _out():
    pltpu.sync_copy(acc.at[pl.ds(s * ZR, ZR)], out_hbm.at[c, pl.ds(s * ZR, ZR)])


def _segsum_call(src2, dst2, table, zeros128):
  return pl.kernel(
      _segsum_body,
      out_type=jax.ShapeDtypeStruct((NC, N, D), jnp.float32),
      mesh=_mesh,
      scratch_types=[
          pltpu.VMEM_SHARED((N, D), jnp.float32),
          pltpu.VMEM((HIDX, CH), jnp.int32),
          pltpu.VMEM((HIDX, CH), jnp.int32),
          pltpu.VMEM((NBUF, CH, D), jnp.float32),
          pltpu.SemaphoreType.DMA((NBUF,)),
          pltpu.SemaphoreType.DMA((NBUF,)),
      ],
  )(src2, dst2, table, zeros128)


R = 1000
_GRID = N // R


def _dis_from(degp_ref):
  deg = degp_ref[0, :, 0] + degp_ref[1, :, 0]
  return jnp.where(deg > 0, lax.rsqrt(deg), 0.0)


def _tc_mm_body(x_ref, w_ref, xw_ref):
  xw_ref[...] = jnp.dot(x_ref[...], w_ref[...],
                        preferred_element_type=jnp.float32)


def _tc_mm(x, w):
  return pl.pallas_call(
      _tc_mm_body,
      grid=(_GRID,),
      in_specs=[
          pl.BlockSpec((R, D), lambda i: (i, 0)),
          pl.BlockSpec((D, D), lambda i: (0, 0)),
      ],
      out_specs=pl.BlockSpec((R, D), lambda i: (i, 0)),
      out_shape=jax.ShapeDtypeStruct((N, D), jnp.float32),
  )(x, w)


def _tc_a_body(degp_ref, x_ref, y1_ref):
  dis = _dis_from(degp_ref)
  y1_ref[...] = x_ref[...] * dis[:, None]


def _tc_a(degp, x):
  return pl.pallas_call(
      _tc_a_body,
      grid=(_GRID,),
      in_specs=[
          pl.BlockSpec((NC, R, D), lambda i: (0, i, 0)),
          pl.BlockSpec((R, D), lambda i: (i, 0)),
      ],
      out_specs=pl.BlockSpec((R, D), lambda i: (i, 0)),
      out_shape=jax.ShapeDtypeStruct((N, D), jnp.float32),
  )(degp, x)


def _tc_b_body(degp_ref, xw_ref, p_ref, w11_ref, b1_ref, g_ref, bb_ref,
               w20_ref, y2_ref, hw_ref):
  dis = _dis_from(degp_ref)
  tx = (p_ref[0] + p_ref[1]) * (-dis[:, None])
  h = (xw_ref[...] + b1_ref[...]
       + jnp.dot(tx, w11_ref[...], preferred_element_type=jnp.float32))
  mu = jnp.mean(h, axis=1, keepdims=True)
  hc = h - mu
  var = jnp.mean(hc * hc, axis=1, keepdims=True)
  hn = hc * lax.rsqrt(var + 1e-5) * g_ref[...] + bb_ref[...]
  h = jnp.where(hn >= 0, hn, 0.01 * hn)
  y2_ref[...] = h * dis[:, None]
  hw_ref[...] = jnp.dot(h, w20_ref[...], preferred_element_type=jnp.float32)


def _tc_b(degp, xw, p, w11, b1, ln_g, ln_b, w20):
  wspec = pl.BlockSpec((D, D), lambda i: (0, 0))
  vspec = pl.BlockSpec((1, D), lambda i: (0, 0))
  rspec = pl.BlockSpec((R, D), lambda i: (i, 0))
  return pl.pallas_call(
      _tc_b_body,
      grid=(_GRID,),
      in_specs=[
          pl.BlockSpec((NC, R, D), lambda i: (0, i, 0)),
          rspec,
          pl.BlockSpec((NC, R, D), lambda i: (0, i, 0)),
          wspec, vspec, vspec, vspec, wspec,
      ],
      out_specs=[rspec, rspec],
      out_shape=[
          jax.ShapeDtypeStruct((N, D), jnp.float32),
          jax.ShapeDtypeStruct((N, D), jnp.float32),
      ],
  )(degp, xw, p, w11, b1.reshape(1, D), ln_g.reshape(1, D),
    ln_b.reshape(1, D), w20)


def _tc_c_body(degp_ref, hw_ref, q_ref, w21_ref, b2_ref, decw_ref, decb_ref,
               pw1_ref, pb1_ref, pw2_ref, pb2_ref, recon_ref, z_ref, proj_ref):
  dis = _dis_from(degp_ref)
  tx = (q_ref[0] + q_ref[1]) * (-dis[:, None])
  z = (hw_ref[...] + b2_ref[...]
       + jnp.dot(tx, w21_ref[...], preferred_element_type=jnp.float32))
  z_ref[...] = z
  recon_ref[...] = jnp.tanh(
      jnp.dot(z, decw_ref[...], preferred_element_type=jnp.float32)
      + decb_ref[...])
  t = jnp.maximum(
      jnp.dot(z, pw1_ref[...], preferred_element_type=jnp.float32)
      + pb1_ref[...], 0.0)
  proj_ref[...] = (
      jnp.dot(t, pw2_ref[...], preferred_element_type=jnp.float32)
      + pb2_ref[...])


def _tc_c(degp, hw, q, w21, b2, dec_w, dec_b, pw1, pb1, pw2, pb2):
  wspec = pl.BlockSpec((D, D), lambda i: (0, 0))
  vspec = pl.BlockSpec((1, D), lambda i: (0, 0))
  rspec = pl.BlockSpec((R, D), lambda i: (i, 0))
  return pl.pallas_call(
      _tc_c_body,
      grid=(_GRID,),
      in_specs=[
          pl.BlockSpec((NC, R, D), lambda i: (0, i, 0)),
          rspec,
          pl.BlockSpec((NC, R, D), lambda i: (0, i, 0)),
          wspec, vspec, wspec, vspec, wspec, vspec, wspec, vspec,
      ],
      out_specs=[rspec, rspec, rspec],
      out_shape=[
          jax.ShapeDtypeStruct((N, D), jnp.float32),
          jax.ShapeDtypeStruct((N, D), jnp.float32),
          jax.ShapeDtypeStruct((N, D), jnp.float32),
      ],
  )(degp, hw, q, w21, b2.reshape(1, D), dec_w, dec_b.reshape(1, D),
    pw1, pb1.reshape(1, D), pw2, pb2.reshape(1, D))


def kernel(x, edge_index, W1_0, W1_1, b1, ln_g, ln_b, W2_0, W2_1, b2,
           dec_W, dec_b, pr_W1, pr_b1, pr_W2, pr_b2):
  src2 = _chunk_layout(edge_index[0].reshape(E // CH, CH))
  dst2 = _chunk_layout(edge_index[1].reshape(E // CH, CH))
  zeros128 = jnp.zeros((ZR, D), jnp.float32)
  ones128 = jnp.ones((CH, D), jnp.float32)

  xw0 = _tc_mm(x, W1_0)
  degp = _deg_call(src2, ones128, zeros128)
  y1 = _tc_a(degp, x)
  p = _segsum_call(src2, dst2, y1, zeros128)
  y2, hw0 = _tc_b(degp, xw0, p, W1_1, b1, ln_g, ln_b, W2_0)
  q = _segsum_call(src2, dst2, y2, zeros128)
  recon, z, proj = _tc_c(degp, hw0, q, W2_1, b2, dec_W, dec_b,
                         pr_W1, pr_b1, pr_W2, pr_b2)
  return (recon, z, proj)
